# Initial kernel scaffold; baseline (speedup 1.0000x reference)
import jax
import jax.numpy as jnp
from jax.experimental import pallas as pl


def kernel(x, edge_index, batch, Wr1, Wo1, b1, Wr2, Wo2, b2, Wr3, Wo3, b3, Wr4, Wo4, b4, Wf1, bf1, Wf2, bf2, Wf3, bf3, Wf4, bf4):
    raise NotImplementedError("write your pallas kernel here")


# speedup vs baseline: 3.6615x; 3.6615x over previous
import functools

import jax
import jax.numpy as jnp
from jax import lax
from jax.experimental import pallas as pl
from jax.experimental.pallas import tpu as pltpu
from jax.experimental.pallas import tpu_sc as plsc

N = 10000
E = 160000
G = 64
F0 = 28
H = 256
HH = 128
NPAD = 10112
RPT = NPAD // 16
NTILES = 16
EPT = E // NTILES
EB = (EPT + 127) // 128
EPT_PAD = EB * 128

_f32 = jnp.float32
_i32 = jnp.int32


@functools.cache
def _get_sc_agg():
    mesh = plsc.VectorSubcoreMesh(core_axis_name="c", subcore_axis_name="s",
                                  num_cores=2, num_subcores=16)
    return functools.partial(
        pl.kernel,
        mesh=mesh,
        out_type=(
            jax.ShapeDtypeStruct((NPAD, HH), _f32),
            jax.ShapeDtypeStruct((NPAD, HH), _f32),
        ),
        scratch_types=[
            pltpu.VMEM((EB, 128), _i32),
            pltpu.VMEM((EB, 128), _i32),
            pltpu.VMEM((128, HH), _f32),
            pltpu.VMEM_SHARED((NPAD, HH), _f32),
            pltpu.SemaphoreType.DMA,
        ],
    )(_sc_agg_body)


def _sc_agg(*args):
    return _get_sc_agg()(*args)


def _sc_agg_body(gA, gB, srcI, dstI, zrows, outA, outB,
                 src_v, dst_v, rows_v, acc, sem):
    c = lax.axis_index("c")
    s = lax.axis_index("s")

    pltpu.sync_copy(srcI.at[s], src_v)
    pltpu.sync_copy(dstI.at[s], dst_v)
    pltpu.sync_copy(zrows, acc.at[pl.ds(s * RPT, RPT)])
    plsc.subcore_barrier()

    def make_body(g_ref):
        def body(j, carry):
            pltpu.async_copy(g_ref.at[src_v.at[j]], rows_v, sem).wait()
            pltpu.sync_copy(rows_v, acc.at[dst_v.at[j]], add=True)
            return carry
        return body

    @pl.when(c == 0)
    def _():
        lax.fori_loop(0, EB, make_body(gA), 0)

    @pl.when(c == 1)
    def _():
        lax.fori_loop(0, EB, make_body(gB), 0)

    plsc.subcore_barrier()

    @pl.when(c == 0)
    def _():
        pltpu.sync_copy(acc.at[pl.ds(s * RPT, RPT)],
                        outA.at[pl.ds(s * RPT, RPT)])

    @pl.when(c == 1)
    def _():
        pltpu.sync_copy(acc.at[pl.ds(s * RPT, RPT)],
                        outB.at[pl.ds(s * RPT, RPT)])


_GRID = NPAD // RPT


def _rowspec():
    return pl.BlockSpec((RPT, HH), lambda i: (i, 0))


def _fullspec(shape):
    return pl.BlockSpec(shape, lambda i: tuple(0 for _ in shape))


def _tc0_body(x_ref, w_ref, gA_ref, gB_ref):
    g = jnp.dot(x_ref[...], w_ref[...], preferred_element_type=_f32)
    gA_ref[...] = g[:, :HH]
    gB_ref[...] = g[:, HH:]


def _tc0(xp, Wr1p):
    return pl.pallas_call(
        _tc0_body,
        grid=(_GRID,),
        in_specs=[pl.BlockSpec((RPT, 32), lambda i: (i, 0)),
                  _fullspec((32, H))],
        out_specs=[_rowspec(), _rowspec()],
        out_shape=[jax.ShapeDtypeStruct((NPAD, HH), _f32)] * 2,
    )(xp, Wr1p)


def _row_mask(i):
    rows = i * RPT + lax.broadcasted_iota(_i32, (RPT, 1), 0)
    return rows < N


def _tc_layer_body(has_next, aA, aB, hA, hB, WoA, WoB, b, Wrn,
                   oA, oB, *g_out):
    i = pl.program_id(0)
    y = jnp.concatenate([aA[...], aB[...]], axis=1)
    y = y + jnp.dot(hA[...], WoA[...], preferred_element_type=_f32)
    y = y + jnp.dot(hB[...], WoB[...], preferred_element_type=_f32)
    y = jnp.maximum(y + b[...], 0.0)
    y = jnp.where(_row_mask(i), y, 0.0)
    oA[...] = y[:, :HH]
    oB[...] = y[:, HH:]
    if has_next:
        g = jnp.dot(y, Wrn[...], preferred_element_type=_f32)
        g_out[0][...] = g[:, :HH]
        g_out[1][...] = g[:, HH:]


def _tc_layer(aA, aB, hA, hB, Wo, b, Wrn):
    has_next = Wrn is not None
    n_out = 4 if has_next else 2
    WoA = Wo[:HH]
    WoB = Wo[HH:]
    b2 = b.reshape(1, H)
    args = [aA, aB, hA, hB, WoA, WoB, b2]
    in_specs = [_rowspec()] * 4 + [_fullspec((HH, H))] * 2 + [_fullspec((1, H))]
    if has_next:
        args.append(Wrn)
        in_specs.append(_fullspec((H, H)))
        body = functools.partial(_tc_layer_body, True)
    else:
        body = lambda *a: _tc_layer_body(False, *a[:7], None, *a[7:])
    return pl.pallas_call(
        body,
        grid=(_GRID,),
        in_specs=in_specs,
        out_specs=[_rowspec()] * n_out,
        out_shape=[jax.ShapeDtypeStruct((NPAD, HH), _f32)] * n_out,
    )(*args)


def _head_body(batch_ref, h1A, h1B, h2A, h2B, h3A, h3B, h4A, h4B,
               Wf1, bf1, Wf2, bf2, Wf3, bf3, Wf4, bf4,
               out_ref, S, C):
    i = pl.program_id(0)

    @pl.when(i == 0)
    def _():
        S[...] = jnp.zeros_like(S)
        C[...] = jnp.zeros_like(C)

    b_ids = batch_ref[0, 0, :]
    onehot = (lax.broadcasted_iota(_i32, (G, RPT), 0)
              == b_ids[None, :]).astype(_f32)
    halves = [h1A, h1B, h2A, h2B, h3A, h3B, h4A, h4B]
    for k, hr in enumerate(halves):
        S[:, k * HH:(k + 1) * HH] += jnp.dot(
            onehot, hr[...], preferred_element_type=_f32)
    C[...] += jnp.broadcast_to(
        jnp.sum(onehot, axis=1, keepdims=True), (G, HH))

    @pl.when(i == _GRID - 1)
    def _():
        cnt = C[:, 0:1]
        pooled = S[...] / jnp.maximum(cnt, 1.0)
        t = jnp.maximum(
            jnp.dot(pooled, Wf1[...], preferred_element_type=_f32)
            + bf1[...], 0.0)
        t = jnp.maximum(
            jnp.dot(t, Wf2[...], preferred_element_type=_f32)
            + bf2[...], 0.0)
        t = jnp.maximum(
            jnp.dot(t, Wf3[...], preferred_element_type=_f32)
            + bf3[...], 0.0)
        out_ref[...] = (jnp.dot(t, Wf4[...], preferred_element_type=_f32)
                        + bf4[...])


def _head(batchp, halves, Wf1, bf1, Wf2, bf2, Wf3, bf3, Wf4, bf4):
    args = [batchp] + list(halves) + [
        Wf1, bf1.reshape(1, H), Wf2, bf2.reshape(1, H),
        Wf3, bf3.reshape(1, H), Wf4, bf4.reshape(1, 1)]
    in_specs = (
        [pl.BlockSpec((1, 1, RPT), lambda i: (i, 0, 0))]
        + [_rowspec()] * 8
        + [_fullspec((4 * H, H)), _fullspec((1, H)),
           _fullspec((H, H)), _fullspec((1, H)),
           _fullspec((H, H)), _fullspec((1, H)),
           _fullspec((H, 1)), _fullspec((1, 1))])
    return pl.pallas_call(
        _head_body,
        grid=(_GRID,),
        in_specs=in_specs,
        out_specs=pl.BlockSpec((G, 1), lambda i: (0, 0)),
        out_shape=jax.ShapeDtypeStruct((G, 1), _f32),
        scratch_shapes=[pltpu.VMEM((G, 4 * H), _f32),
                        pltpu.VMEM((G, HH), _f32)],
    )(*args)


def kernel(x, edge_index, batch,
           Wr1, Wo1, b1, Wr2, Wo2, b2, Wr3, Wo3, b3, Wr4, Wo4, b4,
           Wf1, bf1, Wf2, bf2, Wf3, bf3, Wf4, bf4):
    xp = jnp.zeros((NPAD, 32), _f32).at[:N, :F0].set(x)
    Wr1p = jnp.zeros((32, H), _f32).at[:F0].set(Wr1)
    Wo1p = jnp.zeros((32, H), _f32).at[:F0].set(Wo1)

    src = edge_index[0]
    dst = edge_index[1]
    padi = jnp.full((NTILES, EPT_PAD - EPT), N, _i32)
    srcp = jnp.concatenate([src.reshape(NTILES, EPT), padi],
                           axis=1).reshape(NTILES, EB, 128)
    dstp = jnp.concatenate([dst.reshape(NTILES, EPT), padi],
                           axis=1).reshape(NTILES, EB, 128)
    zrows = jnp.zeros((RPT, HH), _f32)
    batchp = jnp.full((NPAD,), G, _i32).at[:N].set(batch) \
                .reshape(_GRID, 1, RPT)

    g1A, g1B = _tc0(xp, Wr1p)
    a1A, a1B = _sc_agg(g1A, g1B, srcp, dstp, zrows)
    h1A, h1B, g2A, g2B = _tc_layer_l1(a1A, a1B, xp, Wo1p, b1, Wr2)

    a2A, a2B = _sc_agg(g2A, g2B, srcp, dstp, zrows)
    h2A, h2B, g3A, g3B = _tc_layer(a2A, a2B, h1A, h1B, Wo2, b2, Wr3)

    a3A, a3B = _sc_agg(g3A, g3B, srcp, dstp, zrows)
    h3A, h3B, g4A, g4B = _tc_layer(a3A, a3B, h2A, h2B, Wo3, b3, Wr4)

    a4A, a4B = _sc_agg(g4A, g4B, srcp, dstp, zrows)
    h4A, h4B = _tc_layer(a4A, a4B, h3A, h3B, Wo4, b4, None)

    out = _head(batchp, (h1A, h1B, h2A, h2B, h3A, h3B, h4A, h4B),
                Wf1, bf1, Wf2, bf2, Wf3, bf3, Wf4, bf4)
    return out.reshape(-1)


def _tc_l1_body(aA, aB, x_ref, Wo, b, Wrn, oA, oB, gA, gB):
    i = pl.program_id(0)
    y = jnp.concatenate([aA[...], aB[...]], axis=1)
    y = y + jnp.dot(x_ref[...], Wo[...], preferred_element_type=_f32)
    y = jnp.maximum(y + b[...], 0.0)
    y = jnp.where(_row_mask(i), y, 0.0)
    oA[...] = y[:, :HH]
    oB[...] = y[:, HH:]
    g = jnp.dot(y, Wrn[...], preferred_element_type=_f32)
    gA[...] = g[:, :HH]
    gB[...] = g[:, HH:]


def _tc_layer_l1(aA, aB, xp, Wo1p, b1, Wr2):
    return pl.pallas_call(
        _tc_l1_body,
        grid=(_GRID,),
        in_specs=[_rowspec(), _rowspec(),
                  pl.BlockSpec((RPT, 32), lambda i: (i, 0)),
                  _fullspec((32, H)), _fullspec((1, H)), _fullspec((H, H))],
        out_specs=[_rowspec()] * 4,
        out_shape=[jax.ShapeDtypeStruct((NPAD, HH), _f32)] * 4,
    )(aA, aB, xp, Wo1p, b1.reshape(1, H), Wr2)
